# Initial kernel scaffold; baseline (speedup 1.0000x reference)
import jax
import jax.numpy as jnp
from jax.experimental import pallas as pl


def kernel(ids, adj, feats, W1, b1, W2, b2):
    raise NotImplementedError("write your pallas kernel here")


# speedup vs baseline: 2.9819x; 2.9819x over previous
import functools

import jax
import jax.numpy as jnp
import numpy as np
from jax import lax
from jax.experimental import pallas as pl
from jax.experimental.pallas import tpu as pltpu
from jax.experimental.pallas import tpu_sc as plsc

N_NODES = 50000
DEG = 32
D = 256
BATCH = 1024
N_SAMPLE = 16

NC = 2
NS = 16
NW = NC * NS

SEEDS_PER_W = BATCH // NW
J_PER_W = SEEDS_PER_W * N_SAMPLE
CHUNK = 16
N_CHUNKS = J_PER_W // CHUNK

_COLS0 = (17, 27, 1, 3, 28, 19, 9, 11, 31, 5, 15, 20, 0, 14, 2, 21)
_COLS1 = (2, 15, 10, 25, 28, 0, 4, 21, 11, 20, 17, 12, 19, 22, 18, 16)


def _sc_gather_body(ids_h, adjf_h, feats_h,
                    f0_h, f1_h, m2_h,
                    idx0_v, idx1_v, ids1_v, idx2_v, ids2_v,
                    f0_v, f1_v, f2_v, m2_v, sem):
    c = lax.axis_index("c")
    s = lax.axis_index("s")
    wid = s * NC + c
    base0 = wid * SEEDS_PER_W

    pltpu.sync_copy(ids_h.at[pl.ds(base0, SEEDS_PER_W)], idx0_v)
    pltpu.async_copy(feats_h.at[idx0_v], f0_v, sem).wait()
    pltpu.sync_copy(f0_v, f0_h.at[pl.ds(base0, SEEDS_PER_W)])

    for g in range(SEEDS_PER_W // 16):
        seeds = idx0_v[pl.ds(g * 16, 16)] * DEG
        for k in range(N_SAMPLE):
            idx1_v[pl.ds(g * 256 + k * 16, 16)] = seeds + _COLS0[k]
    for q in range(J_PER_W // 128):
        pltpu.async_copy(adjf_h.at[idx1_v.at[pl.ds(q * 128, 128)]],
                         ids1_v.at[pl.ds(q * 128, 128)], sem).wait()

    def chunk_body(cidx, carry):
        nbase = cidx * CHUNK
        gbase = wid * J_PER_W + nbase
        nodes = ids1_v[pl.ds(nbase, CHUNK)] * DEG
        for k in range(N_SAMPLE):
            idx2_v[pl.ds(k * 16, 16)] = nodes + _COLS1[k]
        for q in range(2):
            pltpu.async_copy(adjf_h.at[idx2_v.at[pl.ds(q * 128, 128)]],
                             ids2_v.at[pl.ds(q * 128, 128)], sem).wait()
        pltpu.async_copy(feats_h.at[ids1_v.at[pl.ds(nbase, CHUNK)]], f1_v, sem).wait()
        pltpu.sync_copy(f1_v, f1_h.at[pl.ds(gbase, CHUNK)])
        for q in range(2):
            pltpu.async_copy(feats_h.at[ids2_v.at[pl.ds(q * 128, 128)]],
                             f2_v.at[pl.ds(q * 128, 128)], sem).wait()

        def reduce_node(t, inner):
            for ch in range(D // 16):
                sl = pl.ds(ch * 16, 16)
                acc = f2_v[t, sl]
                for k in range(1, N_SAMPLE):
                    acc = acc + f2_v[k * 16 + t, sl]
                m2_v[t, sl] = acc * (1.0 / N_SAMPLE)
            return inner

        lax.fori_loop(0, CHUNK, reduce_node, 0)
        pltpu.sync_copy(m2_v, m2_h.at[pl.ds(gbase, CHUNK)])
        return carry

    lax.fori_loop(0, N_CHUNKS, chunk_body, 0)


_sc_gather = functools.partial(
    pl.kernel,
    out_type=(
        jax.ShapeDtypeStruct((BATCH, D), jnp.float32),
        jax.ShapeDtypeStruct((BATCH * N_SAMPLE, D), jnp.float32),
        jax.ShapeDtypeStruct((BATCH * N_SAMPLE, D), jnp.float32),
    ),
    mesh=plsc.VectorSubcoreMesh(core_axis_name="c", subcore_axis_name="s"),
    scratch_types=[
        pltpu.VMEM((SEEDS_PER_W,), jnp.int32),
        pltpu.VMEM((J_PER_W,), jnp.int32),
        pltpu.VMEM((J_PER_W,), jnp.int32),
        pltpu.VMEM((CHUNK * N_SAMPLE,), jnp.int32),
        pltpu.VMEM((CHUNK * N_SAMPLE,), jnp.int32),
        pltpu.VMEM((SEEDS_PER_W, D), jnp.float32),
        pltpu.VMEM((CHUNK, D), jnp.float32),
        pltpu.VMEM((CHUNK * N_SAMPLE, D), jnp.float32),
        pltpu.VMEM((CHUNK, D), jnp.float32),
        pltpu.SemaphoreType.DMA,
    ],
)(_sc_gather_body)


G_STEPS = 16
ROWS_PER_STEP = (BATCH * N_SAMPLE) // G_STEPS
GROUPS_PER_STEP = ROWS_PER_STEP // N_SAMPLE


def _group_matrix():
    col = lax.broadcasted_iota(jnp.int32, (GROUPS_PER_STEP, ROWS_PER_STEP), 1)
    seed = (col // 512) * 32 + ((col // 256) % 2) * 16 + (col % 16)
    grp = lax.broadcasted_iota(jnp.int32, (GROUPS_PER_STEP, ROWS_PER_STEP), 0)
    return jnp.where(seed == grp, 1.0 / N_SAMPLE, 0.0).astype(jnp.float32)


def _tc_dense_body(f0_ref, f1_ref, m2_ref, w1a_ref, w1b_ref, b1_ref,
                   w2a_ref, w2b_ref, b2_ref, out_ref, m1_acc, mh1_acc):
    step = pl.program_id(0)

    @pl.when(step < G_STEPS)
    def _phase1():
        f1c = f1_ref[...]
        m2c = m2_ref[...]
        h1 = jnp.dot(f1c, w1a_ref[...], preferred_element_type=jnp.float32)
        h1 = h1 + jnp.dot(m2c, w1b_ref[...], preferred_element_type=jnp.float32)
        h1 = jnp.maximum(h1 + b1_ref[...], 0.0)
        a = _group_matrix()
        m1_acc[pl.ds(step * GROUPS_PER_STEP, GROUPS_PER_STEP), :] = jnp.dot(
            a, f1c, preferred_element_type=jnp.float32)
        mh1_acc[pl.ds(step * GROUPS_PER_STEP, GROUPS_PER_STEP), :] = jnp.dot(
            a, h1, preferred_element_type=jnp.float32)

    @pl.when(step == G_STEPS)
    def _phase2():
        f0 = f0_ref[...]
        h0 = jnp.dot(f0, w1a_ref[...], preferred_element_type=jnp.float32)
        h0 = h0 + jnp.dot(m1_acc[...], w1b_ref[...], preferred_element_type=jnp.float32)
        h0 = jnp.maximum(h0 + b1_ref[...], 0.0)
        out = jnp.dot(h0, w2a_ref[...], preferred_element_type=jnp.float32)
        out = out + jnp.dot(mh1_acc[...], w2b_ref[...], preferred_element_type=jnp.float32)
        out_ref[...] = out + b2_ref[...]


def _tc_dense(f0, f1, m2, w1a, w1b, b1, w2a, w2b, b2):
    return pl.pallas_call(
        _tc_dense_body,
        grid=(G_STEPS + 1,),
        in_specs=[
            pl.BlockSpec((BATCH, D), lambda i: (0, 0)),
            pl.BlockSpec((ROWS_PER_STEP, D), lambda i: (jnp.minimum(i, G_STEPS - 1), 0)),
            pl.BlockSpec((ROWS_PER_STEP, D), lambda i: (jnp.minimum(i, G_STEPS - 1), 0)),
            pl.BlockSpec((D, D), lambda i: (0, 0)),
            pl.BlockSpec((D, D), lambda i: (0, 0)),
            pl.BlockSpec((1, D), lambda i: (0, 0)),
            pl.BlockSpec((D, D), lambda i: (0, 0)),
            pl.BlockSpec((D, D), lambda i: (0, 0)),
            pl.BlockSpec((1, D), lambda i: (0, 0)),
        ],
        out_specs=pl.BlockSpec((BATCH, D), lambda i: (0, 0)),
        out_shape=jax.ShapeDtypeStruct((BATCH, D), jnp.float32),
        scratch_shapes=[
            pltpu.VMEM((BATCH, D), jnp.float32),
            pltpu.VMEM((BATCH, D), jnp.float32),
        ],
    )(f0, f1, m2, w1a, w1b, b1, w2a, w2b, b2)


def kernel(ids, adj, feats, W1, b1, W2, b2):
    f0, f1, m2 = _sc_gather(ids, adj.reshape(-1), feats)
    out = _tc_dense(
        f0, f1, m2,
        W1[:D], W1[D:], b1.reshape(1, D),
        W2[:D], W2[D:], b2.reshape(1, D),
    )
    return out


# speedup vs baseline: 4.9215x; 1.6505x over previous
import functools

import jax
import jax.numpy as jnp
import numpy as np
from jax import lax
from jax.experimental import pallas as pl
from jax.experimental.pallas import tpu as pltpu
from jax.experimental.pallas import tpu_sc as plsc

N_NODES = 50000
DEG = 32
D = 256
BATCH = 1024
N_SAMPLE = 16

NC = 2
NS = 16
NW = NC * NS

SEEDS_PER_W = BATCH // NW
J_PER_W = SEEDS_PER_W * N_SAMPLE
CHUNK = 16
N_CHUNKS = J_PER_W // CHUNK

_COLS0 = (17, 27, 1, 3, 28, 19, 9, 11, 31, 5, 15, 20, 0, 14, 2, 21)
_COLS1 = (2, 15, 10, 25, 28, 0, 4, 21, 11, 20, 17, 12, 19, 22, 18, 16)


N_GROUPS = J_PER_W // CHUNK
HALF = CHUNK * N_SAMPLE // 2


def _sc_gather_body(ids_h, adjf_h, feats_h,
                    f0_h, f1_h, m2_h,
                    idx0_v, idx1_v, ids1_v, idx2_v, ids2_v,
                    f0_v, f1_v, f2a_v, f2b_v, m2_v,
                    sem0, sem1, sem_f1, sem_ids, sem_aux):
    c = lax.axis_index("c")
    s = lax.axis_index("s")
    wid = s * NC + c
    base0 = wid * SEEDS_PER_W

    pltpu.sync_copy(ids_h.at[pl.ds(base0, SEEDS_PER_W)], idx0_v)
    f0_dma = pltpu.async_copy(feats_h.at[idx0_v], f0_v, sem_aux)

    for g in range(SEEDS_PER_W // 16):
        seeds = idx0_v[pl.ds(g * 16, 16)] * DEG
        for k in range(N_SAMPLE):
            idx1_v[pl.ds(g * 256 + k * 16, 16)] = seeds + _COLS0[k]
    for q in range(J_PER_W // 128):
        pltpu.async_copy(adjf_h.at[idx1_v.at[pl.ds(q * 128, 128)]],
                         ids1_v.at[pl.ds(q * 128, 128)], sem_ids)
    pltpu.make_async_copy(adjf_h.at[pl.ds(0, J_PER_W)], ids1_v, sem_ids).wait()

    def build_idx2(g, carry):
        nodes = ids1_v[pl.ds(g * CHUNK, CHUNK)] * DEG
        for k in range(N_SAMPLE):
            idx2_v[pl.ds(g * 256 + k * 16, 16)] = nodes + _COLS1[k]
        return carry

    lax.fori_loop(0, N_GROUPS, build_idx2, 0)

    def fire_ids2(q, carry):
        pltpu.async_copy(adjf_h.at[idx2_v.at[pl.ds(q * 128, 128)]],
                         ids2_v.at[pl.ds(q * 128, 128)], sem_ids)
        return carry

    lax.fori_loop(0, N_GROUPS * 2, fire_ids2, 0)
    pltpu.make_async_copy(adjf_h.at[pl.ds(0, N_GROUPS * 256)], ids2_v, sem_ids).wait()

    f0_dma.wait()
    pltpu.sync_copy(f0_v, f0_h.at[pl.ds(base0, SEEDS_PER_W)])

    def _fire_half(g, h, buf, sem):
        pltpu.async_copy(
            feats_h.at[ids2_v.at[pl.ds(g * 256 + h * HALF, HALF)]], buf, sem)

    def _wait_half(buf, sem):
        pltpu.make_async_copy(feats_h.at[pl.ds(0, HALF)], buf, sem).wait()

    def _fire_f1(g):
        pltpu.async_copy(feats_h.at[ids1_v.at[pl.ds(g * CHUNK, CHUNK)]],
                         f1_v, sem_f1)

    _fire_half(0, 0, f2a_v, sem0)
    _fire_half(0, 1, f2b_v, sem1)
    _fire_f1(0)

    def _reduce_half(src, first):
        def reduce_node(t, inner):
            for ch in range(D // 16):
                sl = pl.ds(ch * 16, 16)
                acc = src[t, sl]
                for kl in range(1, N_SAMPLE // 2):
                    acc = acc + src[kl * 16 + t, sl]
                if first:
                    m2_v[t, sl] = acc
                else:
                    m2_v[t, sl] = (m2_v[t, sl] + acc) * (1.0 / N_SAMPLE)
            return inner

        lax.fori_loop(0, CHUNK, reduce_node, 0)

    def group_body(g, carry):
        gbase = wid * J_PER_W + g * CHUNK
        _wait_half(f2a_v, sem0)
        _reduce_half(f2a_v, True)

        @pl.when(g < N_GROUPS - 1)
        def _():
            _fire_half(g + 1, 0, f2a_v, sem0)

        _wait_half(f2b_v, sem1)
        _reduce_half(f2b_v, False)

        @pl.when(g < N_GROUPS - 1)
        def _():
            _fire_half(g + 1, 1, f2b_v, sem1)

        pltpu.sync_copy(m2_v, m2_h.at[pl.ds(gbase, CHUNK)])

        pltpu.make_async_copy(feats_h.at[pl.ds(0, CHUNK)], f1_v, sem_f1).wait()
        pltpu.sync_copy(f1_v, f1_h.at[pl.ds(gbase, CHUNK)])

        @pl.when(g < N_GROUPS - 1)
        def _():
            _fire_f1(g + 1)

        return carry

    lax.fori_loop(0, N_GROUPS, group_body, 0)


_sc_gather = functools.partial(
    pl.kernel,
    out_type=(
        jax.ShapeDtypeStruct((BATCH, D), jnp.float32),
        jax.ShapeDtypeStruct((BATCH * N_SAMPLE, D), jnp.float32),
        jax.ShapeDtypeStruct((BATCH * N_SAMPLE, D), jnp.float32),
    ),
    mesh=plsc.VectorSubcoreMesh(core_axis_name="c", subcore_axis_name="s"),
    scratch_types=[
        pltpu.VMEM((SEEDS_PER_W,), jnp.int32),
        pltpu.VMEM((J_PER_W,), jnp.int32),
        pltpu.VMEM((J_PER_W,), jnp.int32),
        pltpu.VMEM((J_PER_W * N_SAMPLE,), jnp.int32),
        pltpu.VMEM((J_PER_W * N_SAMPLE,), jnp.int32),
        pltpu.VMEM((SEEDS_PER_W, D), jnp.float32),
        pltpu.VMEM((CHUNK, D), jnp.float32),
        pltpu.VMEM((HALF, D), jnp.float32),
        pltpu.VMEM((HALF, D), jnp.float32),
        pltpu.VMEM((CHUNK, D), jnp.float32),
        pltpu.SemaphoreType.DMA,
        pltpu.SemaphoreType.DMA,
        pltpu.SemaphoreType.DMA,
        pltpu.SemaphoreType.DMA,
        pltpu.SemaphoreType.DMA,
    ],
)(_sc_gather_body)


G_STEPS = 16
ROWS_PER_STEP = (BATCH * N_SAMPLE) // G_STEPS
GROUPS_PER_STEP = ROWS_PER_STEP // N_SAMPLE


def _group_matrix():
    col = lax.broadcasted_iota(jnp.int32, (GROUPS_PER_STEP, ROWS_PER_STEP), 1)
    seed = (col // 512) * 32 + ((col // 256) % 2) * 16 + (col % 16)
    grp = lax.broadcasted_iota(jnp.int32, (GROUPS_PER_STEP, ROWS_PER_STEP), 0)
    return jnp.where(seed == grp, 1.0 / N_SAMPLE, 0.0).astype(jnp.float32)


def _tc_dense_body(f0_ref, f1_ref, m2_ref, w1a_ref, w1b_ref, b1_ref,
                   w2a_ref, w2b_ref, b2_ref, out_ref, m1_acc, mh1_acc):
    step = pl.program_id(0)

    @pl.when(step < G_STEPS)
    def _phase1():
        f1c = f1_ref[...]
        m2c = m2_ref[...]
        h1 = jnp.dot(f1c, w1a_ref[...], preferred_element_type=jnp.float32)
        h1 = h1 + jnp.dot(m2c, w1b_ref[...], preferred_element_type=jnp.float32)
        h1 = jnp.maximum(h1 + b1_ref[...], 0.0)
        a = _group_matrix()
        m1_acc[pl.ds(step * GROUPS_PER_STEP, GROUPS_PER_STEP), :] = jnp.dot(
            a, f1c, preferred_element_type=jnp.float32)
        mh1_acc[pl.ds(step * GROUPS_PER_STEP, GROUPS_PER_STEP), :] = jnp.dot(
            a, h1, preferred_element_type=jnp.float32)

    @pl.when(step == G_STEPS)
    def _phase2():
        f0 = f0_ref[...]
        h0 = jnp.dot(f0, w1a_ref[...], preferred_element_type=jnp.float32)
        h0 = h0 + jnp.dot(m1_acc[...], w1b_ref[...], preferred_element_type=jnp.float32)
        h0 = jnp.maximum(h0 + b1_ref[...], 0.0)
        out = jnp.dot(h0, w2a_ref[...], preferred_element_type=jnp.float32)
        out = out + jnp.dot(mh1_acc[...], w2b_ref[...], preferred_element_type=jnp.float32)
        out_ref[...] = out + b2_ref[...]


def _tc_dense(f0, f1, m2, w1a, w1b, b1, w2a, w2b, b2):
    return pl.pallas_call(
        _tc_dense_body,
        grid=(G_STEPS + 1,),
        in_specs=[
            pl.BlockSpec((BATCH, D), lambda i: (0, 0)),
            pl.BlockSpec((ROWS_PER_STEP, D), lambda i: (jnp.minimum(i, G_STEPS - 1), 0)),
            pl.BlockSpec((ROWS_PER_STEP, D), lambda i: (jnp.minimum(i, G_STEPS - 1), 0)),
            pl.BlockSpec((D, D), lambda i: (0, 0)),
            pl.BlockSpec((D, D), lambda i: (0, 0)),
            pl.BlockSpec((1, D), lambda i: (0, 0)),
            pl.BlockSpec((D, D), lambda i: (0, 0)),
            pl.BlockSpec((D, D), lambda i: (0, 0)),
            pl.BlockSpec((1, D), lambda i: (0, 0)),
        ],
        out_specs=pl.BlockSpec((BATCH, D), lambda i: (0, 0)),
        out_shape=jax.ShapeDtypeStruct((BATCH, D), jnp.float32),
        scratch_shapes=[
            pltpu.VMEM((BATCH, D), jnp.float32),
            pltpu.VMEM((BATCH, D), jnp.float32),
        ],
    )(f0, f1, m2, w1a, w1b, b1, w2a, w2b, b2)


def kernel(ids, adj, feats, W1, b1, W2, b2):
    f0, f1, m2 = _sc_gather(ids, adj.reshape(-1), feats)
    out = _tc_dense(
        f0, f1, m2,
        W1[:D], W1[D:], b1.reshape(1, D),
        W2[:D], W2[D:], b2.reshape(1, D),
    )
    return out


# speedup vs baseline: 5.1437x; 1.0451x over previous
import functools

import jax
import jax.numpy as jnp
import numpy as np
from jax import lax
from jax.experimental import pallas as pl
from jax.experimental.pallas import tpu as pltpu
from jax.experimental.pallas import tpu_sc as plsc

N_NODES = 50000
DEG = 32
D = 256
BATCH = 1024
N_SAMPLE = 16

NC = 2
NS = 16
NW = NC * NS

SEEDS_PER_W = BATCH // NW
J_PER_W = SEEDS_PER_W * N_SAMPLE
CHUNK = 16
N_CHUNKS = J_PER_W // CHUNK

_COLS0 = (17, 27, 1, 3, 28, 19, 9, 11, 31, 5, 15, 20, 0, 14, 2, 21)
_COLS1 = (2, 15, 10, 25, 28, 0, 4, 21, 11, 20, 17, 12, 19, 22, 18, 16)


N_GROUPS = J_PER_W // CHUNK
HALF = CHUNK * N_SAMPLE // 2


def _sc_gather_body(ids_h, adjf_h, feats_h,
                    f0_h, f1_h, m2_h,
                    idx0_v, idx1_v, ids1_v, idx2_v, ids2_v,
                    f0_v, f1a_v, f1b_v, f2a_v, f2b_v, m2a_v, m2b_v,
                    sem0, sem1, sem_f1, sem_ids, sem_aux, sem_wm, sem_wf):
    c = lax.axis_index("c")
    s = lax.axis_index("s")
    wid = s * NC + c
    base0 = wid * SEEDS_PER_W

    pltpu.sync_copy(ids_h.at[pl.ds(base0, SEEDS_PER_W)], idx0_v)
    f0_dma = pltpu.async_copy(feats_h.at[idx0_v], f0_v, sem_aux)

    for g in range(SEEDS_PER_W // 16):
        seeds = idx0_v[pl.ds(g * 16, 16)] * DEG
        for k in range(N_SAMPLE):
            idx1_v[pl.ds(g * 256 + k * 16, 16)] = seeds + _COLS0[k]
    for q in range(J_PER_W // 128):
        pltpu.async_copy(adjf_h.at[idx1_v.at[pl.ds(q * 128, 128)]],
                         ids1_v.at[pl.ds(q * 128, 128)], sem_ids)
    pltpu.make_async_copy(adjf_h.at[pl.ds(0, J_PER_W)], ids1_v, sem_ids).wait()

    def build_idx2(g, carry):
        nodes = ids1_v[pl.ds(g * CHUNK, CHUNK)] * DEG
        for k in range(N_SAMPLE):
            idx2_v[pl.ds(g * 256 + k * 16, 16)] = nodes + _COLS1[k]
        return carry

    lax.fori_loop(0, N_GROUPS, build_idx2, 0)

    def fire_ids2(q, carry):
        pltpu.async_copy(adjf_h.at[idx2_v.at[pl.ds(q * 128, 128)]],
                         ids2_v.at[pl.ds(q * 128, 128)], sem_ids)
        return carry

    lax.fori_loop(0, N_GROUPS * 2, fire_ids2, 0)
    pltpu.make_async_copy(adjf_h.at[pl.ds(0, N_GROUPS * 256)], ids2_v, sem_ids).wait()

    f0_dma.wait()
    pltpu.sync_copy(f0_v, f0_h.at[pl.ds(base0, SEEDS_PER_W)])

    def _fire_half(g, h, buf, sem):
        pltpu.async_copy(
            feats_h.at[ids2_v.at[pl.ds(g * 256 + h * HALF, HALF)]], buf, sem)

    def _wait_half(buf, sem):
        pltpu.make_async_copy(feats_h.at[pl.ds(0, HALF)], buf, sem).wait()

    def _fire_f1(g, buf):
        pltpu.async_copy(feats_h.at[ids1_v.at[pl.ds(g * CHUNK, CHUNK)]],
                         buf, sem_f1)

    _fire_half(0, 0, f2a_v, sem0)
    _fire_half(0, 1, f2b_v, sem1)
    _fire_f1(0, f1a_v)

    def _reduce_half(src, dst, first):
        def reduce_node(t, inner):
            for ch in range(D // 16):
                sl = pl.ds(ch * 16, 16)
                acc = src[t, sl]
                for kl in range(1, N_SAMPLE // 2):
                    acc = acc + src[kl * 16 + t, sl]
                if first:
                    dst[t, sl] = acc
                else:
                    dst[t, sl] = (dst[t, sl] + acc) * (1.0 / N_SAMPLE)
            return inner

        lax.fori_loop(0, CHUNK, reduce_node, 0)

    def _wait_wb(buf, sem):
        pltpu.make_async_copy(buf, f1_h.at[pl.ds(0, CHUNK)], sem).wait()

    def group_body_p(g, m2_p, f1_p, f1_q):
        gbase = wid * J_PER_W + g * CHUNK
        _wait_half(f2a_v, sem0)
        @pl.when(g >= 2)
        def _():
            _wait_wb(m2_p, sem_wm)

        _reduce_half(f2a_v, m2_p, True)

        @pl.when(g < N_GROUPS - 1)
        def _():
            _fire_half(g + 1, 0, f2a_v, sem0)

        _wait_half(f2b_v, sem1)
        _reduce_half(f2b_v, m2_p, False)

        @pl.when(g < N_GROUPS - 1)
        def _():
            _fire_half(g + 1, 1, f2b_v, sem1)

        pltpu.async_copy(m2_p, m2_h.at[pl.ds(gbase, CHUNK)], sem_wm)

        pltpu.make_async_copy(feats_h.at[pl.ds(0, CHUNK)], f1_p, sem_f1).wait()
        pltpu.async_copy(f1_p, f1_h.at[pl.ds(gbase, CHUNK)], sem_wf)

        @pl.when(g < N_GROUPS - 1)
        def _():
            @pl.when(g >= 1)
            def _():
                _wait_wb(f1_q, sem_wf)
            _fire_f1(g + 1, f1_q)

    def group_body(i, carry):
        g = i * 2
        group_body_p(g, m2a_v, f1a_v, f1b_v)
        group_body_p(g + 1, m2b_v, f1b_v, f1a_v)
        return carry

    lax.fori_loop(0, N_GROUPS // 2, group_body, 0)
    _wait_wb(m2a_v, sem_wm)
    _wait_wb(m2b_v, sem_wm)
    _wait_wb(f1a_v, sem_wf)
    _wait_wb(f1b_v, sem_wf)


_sc_gather = functools.partial(
    pl.kernel,
    out_type=(
        jax.ShapeDtypeStruct((BATCH, D), jnp.float32),
        jax.ShapeDtypeStruct((BATCH * N_SAMPLE, D), jnp.float32),
        jax.ShapeDtypeStruct((BATCH * N_SAMPLE, D), jnp.float32),
    ),
    mesh=plsc.VectorSubcoreMesh(core_axis_name="c", subcore_axis_name="s"),
    scratch_types=[
        pltpu.VMEM((SEEDS_PER_W,), jnp.int32),
        pltpu.VMEM((J_PER_W,), jnp.int32),
        pltpu.VMEM((J_PER_W,), jnp.int32),
        pltpu.VMEM((J_PER_W * N_SAMPLE,), jnp.int32),
        pltpu.VMEM((J_PER_W * N_SAMPLE,), jnp.int32),
        pltpu.VMEM((SEEDS_PER_W, D), jnp.float32),
        pltpu.VMEM((CHUNK, D), jnp.float32),
        pltpu.VMEM((CHUNK, D), jnp.float32),
        pltpu.VMEM((HALF, D), jnp.float32),
        pltpu.VMEM((HALF, D), jnp.float32),
        pltpu.VMEM((CHUNK, D), jnp.float32),
        pltpu.VMEM((CHUNK, D), jnp.float32),
        pltpu.SemaphoreType.DMA,
        pltpu.SemaphoreType.DMA,
        pltpu.SemaphoreType.DMA,
        pltpu.SemaphoreType.DMA,
        pltpu.SemaphoreType.DMA,
        pltpu.SemaphoreType.DMA,
        pltpu.SemaphoreType.DMA,
    ],
)(_sc_gather_body)


G_STEPS = 16
ROWS_PER_STEP = (BATCH * N_SAMPLE) // G_STEPS
GROUPS_PER_STEP = ROWS_PER_STEP // N_SAMPLE


def _group_matrix():
    col = lax.broadcasted_iota(jnp.int32, (GROUPS_PER_STEP, ROWS_PER_STEP), 1)
    seed = (col // 512) * 32 + ((col // 256) % 2) * 16 + (col % 16)
    grp = lax.broadcasted_iota(jnp.int32, (GROUPS_PER_STEP, ROWS_PER_STEP), 0)
    return jnp.where(seed == grp, 1.0 / N_SAMPLE, 0.0).astype(jnp.float32)


def _tc_dense_body(f0_ref, f1_ref, m2_ref, w1a_ref, w1b_ref, b1_ref,
                   w2a_ref, w2b_ref, b2_ref, out_ref, m1_acc, mh1_acc):
    step = pl.program_id(0)

    @pl.when(step < G_STEPS)
    def _phase1():
        f1c = f1_ref[...]
        m2c = m2_ref[...]
        h1 = jnp.dot(f1c, w1a_ref[...], preferred_element_type=jnp.float32)
        h1 = h1 + jnp.dot(m2c, w1b_ref[...], preferred_element_type=jnp.float32)
        h1 = jnp.maximum(h1 + b1_ref[...], 0.0)
        a = _group_matrix()
        m1_acc[pl.ds(step * GROUPS_PER_STEP, GROUPS_PER_STEP), :] = jnp.dot(
            a, f1c, preferred_element_type=jnp.float32)
        mh1_acc[pl.ds(step * GROUPS_PER_STEP, GROUPS_PER_STEP), :] = jnp.dot(
            a, h1, preferred_element_type=jnp.float32)

    @pl.when(step == G_STEPS)
    def _phase2():
        f0 = f0_ref[...]
        h0 = jnp.dot(f0, w1a_ref[...], preferred_element_type=jnp.float32)
        h0 = h0 + jnp.dot(m1_acc[...], w1b_ref[...], preferred_element_type=jnp.float32)
        h0 = jnp.maximum(h0 + b1_ref[...], 0.0)
        out = jnp.dot(h0, w2a_ref[...], preferred_element_type=jnp.float32)
        out = out + jnp.dot(mh1_acc[...], w2b_ref[...], preferred_element_type=jnp.float32)
        out_ref[...] = out + b2_ref[...]


def _tc_dense(f0, f1, m2, w1a, w1b, b1, w2a, w2b, b2):
    return pl.pallas_call(
        _tc_dense_body,
        grid=(G_STEPS + 1,),
        in_specs=[
            pl.BlockSpec((BATCH, D), lambda i: (0, 0)),
            pl.BlockSpec((ROWS_PER_STEP, D), lambda i: (jnp.minimum(i, G_STEPS - 1), 0)),
            pl.BlockSpec((ROWS_PER_STEP, D), lambda i: (jnp.minimum(i, G_STEPS - 1), 0)),
            pl.BlockSpec((D, D), lambda i: (0, 0)),
            pl.BlockSpec((D, D), lambda i: (0, 0)),
            pl.BlockSpec((1, D), lambda i: (0, 0)),
            pl.BlockSpec((D, D), lambda i: (0, 0)),
            pl.BlockSpec((D, D), lambda i: (0, 0)),
            pl.BlockSpec((1, D), lambda i: (0, 0)),
        ],
        out_specs=pl.BlockSpec((BATCH, D), lambda i: (0, 0)),
        out_shape=jax.ShapeDtypeStruct((BATCH, D), jnp.float32),
        scratch_shapes=[
            pltpu.VMEM((BATCH, D), jnp.float32),
            pltpu.VMEM((BATCH, D), jnp.float32),
        ],
    )(f0, f1, m2, w1a, w1b, b1, w2a, w2b, b2)


def kernel(ids, adj, feats, W1, b1, W2, b2):
    f0, f1, m2 = _sc_gather(ids, adj.reshape(-1), feats)
    out = _tc_dense(
        f0, f1, m2,
        W1[:D], W1[D:], b1.reshape(1, D),
        W2[:D], W2[D:], b2.reshape(1, D),
    )
    return out
